# baseline (speedup 1.0000x reference)
import functools

import jax
import jax.numpy as jnp
from jax import lax
from jax.experimental import pallas as pl
from jax.experimental.pallas import tpu as pltpu
from jax.experimental.pallas import tpu_sc as plsc

N = 10000
E = 320000
D = 128
H = 128
Z = 64
G = 64

NC = 2
NS = 16
EPT = E // NS
C = 20
NB = 8
KCH = 32
EPT_PAD = 20480
NBLK = EPT_PAD // (C * KCH)
NP = 10240
RPT = NP // NS


def _agg_body(feat, srcs, dsts, out, idx_sa, idx_da, idx_sb, idx_db,
              r0, r1, r2, r3, r4, r5, r6, r7, acc,
              g0, g1, g2, g3, g4, g5, g6, g7,
              t0, t1, t2, t3, t4, t5, t6, t7,
              semia, semib):
    c = lax.axis_index("c")
    s = lax.axis_index("s")
    rows = (r0, r1, r2, r3, r4, r5, r6, r7)
    semg = (g0, g1, g2, g3, g4, g5, g6, g7)
    sems = (t0, t1, t2, t3, t4, t5, t6, t7)

    def zrow(i, carry):
        for j in range(H // 16):
            r0[i, pl.ds(16 * j, 16)] = jnp.zeros((16,), jnp.float32)
        return carry

    lax.fori_loop(0, C, zrow, 0)

    def zchunk(k, carry):
        pltpu.sync_copy(r0, acc.at[pl.ds(s * RPT + k * C, C)])
        return carry

    lax.fori_loop(0, RPT // C, zchunk, 0)
    plsc.subcore_barrier()

    def wait_gather(buf, sem):
        pltpu.make_async_copy(feat.at[idx_sa.at[0]], buf, sem).wait()

    def wait_scatter(buf, sem):
        pltpu.make_async_copy(buf, acc.at[idx_da.at[0]], sem).wait()

    def load_idx(kk, bufs, sem):
        pltpu.async_copy(srcs.at[c, s, kk], bufs[0], sem)
        pltpu.async_copy(dsts.at[c, s, kk], bufs[1], sem)

    def wait_idx(bufs, sem):
        pltpu.make_async_copy(srcs.at[c, s, 0], bufs[0], sem).wait()
        pltpu.make_async_copy(dsts.at[c, s, 0], bufs[1], sem).wait()

    def process_block(cur, nxt, nxt_sem):
        cs, cd = cur
        def quad(q, carry2):
            k = NB * q
            for j in range(NB):
                wait_gather(rows[j], semg[j])
                pltpu.async_copy(rows[j], acc.at[cd.at[k + j]], sems[j], add=True)
            for j in range(NB):
                wait_scatter(rows[j], sems[j])
                pltpu.async_copy(feat.at[cs.at[k + NB + j]], rows[j], semg[j])
            return carry2

        lax.fori_loop(0, KCH // NB - 1, quad, 0)
        k = KCH - NB
        for j in range(NB):
            wait_gather(rows[j], semg[j])
            pltpu.async_copy(rows[j], acc.at[cd.at[k + j]], sems[j], add=True)
        if nxt is None:
            for j in range(NB):
                wait_scatter(rows[j], sems[j])
        else:
            wait_idx(nxt, nxt_sem)
            for j in range(NB):
                wait_scatter(rows[j], sems[j])
                pltpu.async_copy(feat.at[nxt[0].at[j]], rows[j], semg[j])

    pltpu.sync_copy(srcs.at[c, s, 0], idx_sa)
    pltpu.sync_copy(dsts.at[c, s, 0], idx_da)
    for j in range(NB):
        pltpu.async_copy(feat.at[idx_sa.at[j]], rows[j], semg[j])
    buf_a = (idx_sa, idx_da)
    buf_b = (idx_sb, idx_db)

    def superblock(sb, carry):
        kk = 2 * sb
        load_idx(kk + 1, buf_b, semib)
        process_block(buf_a, buf_b, semib)
        load_idx(kk + 2, buf_a, semia)
        process_block(buf_b, buf_a, semia)
        return carry

    lax.fori_loop(0, NBLK // 2 - 1, superblock, 0)
    load_idx(NBLK - 1, buf_b, semib)
    process_block(buf_a, buf_b, semib)
    process_block(buf_b, None, None)
    plsc.subcore_barrier()

    pltpu.sync_copy(acc.at[pl.ds(s * RPT, RPT)], out.at[c, pl.ds(s * RPT, RPT)])


@functools.cache
def _make_agg_kernel():
    return functools.partial(
        pl.kernel,
        out_type=jax.ShapeDtypeStruct((2, NP, H), jnp.float32),
        mesh=plsc.VectorSubcoreMesh(core_axis_name="c", subcore_axis_name="s"),
        scratch_types=[
            pltpu.VMEM((KCH, C), jnp.int32),
            pltpu.VMEM((KCH, C), jnp.int32),
            pltpu.VMEM((KCH, C), jnp.int32),
            pltpu.VMEM((KCH, C), jnp.int32),
            *[pltpu.VMEM((C, H), jnp.float32) for _ in range(NB)],
            pltpu.VMEM_SHARED((NP, H), jnp.float32),
            *[pltpu.SemaphoreType.DMA for _ in range(2 * NB + 2)],
        ],
    )(_agg_body)


def _gin_mlp(f, a, w1, b1, g, bb, w2, b2, final_relu):
    h = f + a
    h = jnp.dot(h, w1, preferred_element_type=jnp.float32) + b1
    m = jnp.mean(h, axis=0, keepdims=True)
    v = jnp.mean((h - m) ** 2, axis=0, keepdims=True)
    h = (h - m) * lax.rsqrt(v + 1e-5) * g + bb
    h = jnp.maximum(h, 0.0)
    h = jnp.dot(h, w2, preferred_element_type=jnp.float32) + b2
    if final_relu:
        h = jnp.maximum(h, 0.0)
    return h


def _dense12_body(feat, agg, w1, b1, g, bb, w2, b2, fwa, fwb, fb, out):
    f = feat[...]
    ha = _gin_mlp(f, agg[0, :N], w1[...], b1[...], g[...], bb[...], w2[...], b2[...], True)
    hb = _gin_mlp(f, agg[1, :N], w1[...], b1[...], g[...], bb[...], w2[...], b2[...], True)
    h = (jnp.dot(ha, fwa[...], preferred_element_type=jnp.float32)
         + jnp.dot(hb, fwb[...], preferred_element_type=jnp.float32) + fb[...])
    out[...] = jnp.maximum(h, 0.0)


_dense12 = pl.pallas_call(
    _dense12_body,
    out_shape=jax.ShapeDtypeStruct((N, H), jnp.float32),
)


def _dense3_body(h1r, h2r, agg, batch, w1, b1, g, bb, w2, b2,
                 f3a, f3b, f3bb, f4a, f4b, f4c, f4bb, f5a, f5b, f5c, f5bb,
                 zg_out, h_out):
    h1 = h1r[...]
    h2 = h2r[...]
    ha = _gin_mlp(h2, agg[0, :N], w1[...], b1[...], g[...], bb[...], w2[...], b2[...], False)
    hb = _gin_mlp(h2, agg[1, :N], w1[...], b1[...], g[...], bb[...], w2[...], b2[...], False)
    h3 = jnp.maximum(
        jnp.dot(ha, f3a[...], preferred_element_type=jnp.float32)
        + jnp.dot(hb, f3b[...], preferred_element_type=jnp.float32) + f3bb[...], 0.0)

    sel = (lax.broadcasted_iota(jnp.int32, (G, N), 0) == batch[...]).astype(jnp.float32)
    p1 = jnp.dot(sel, h1, preferred_element_type=jnp.float32)
    p2 = jnp.dot(sel, h2, preferred_element_type=jnp.float32)
    p3 = jnp.dot(sel, h3, preferred_element_type=jnp.float32)
    zg = (jnp.dot(jnp.maximum(p1, 0.0), f4a[...], preferred_element_type=jnp.float32)
          + jnp.dot(jnp.maximum(p2, 0.0), f4b[...], preferred_element_type=jnp.float32)
          + jnp.dot(jnp.maximum(p3, 0.0), f4c[...], preferred_element_type=jnp.float32)
          + f4bb[...])
    zg_out[...] = zg
    h_out[...] = (jnp.dot(h1, f5a[...], preferred_element_type=jnp.float32)
                  + jnp.dot(h2, f5b[...], preferred_element_type=jnp.float32)
                  + jnp.dot(h3, f5c[...], preferred_element_type=jnp.float32)
                  + f5bb[...])


_dense3 = pl.pallas_call(
    _dense3_body,
    out_shape=(jax.ShapeDtypeStruct((G, Z), jnp.float32),
               jax.ShapeDtypeStruct((N, Z), jnp.float32)),
)


def _row(v):
    return v.reshape(1, -1)


def kernel(x, params, edge_index, edge_index_t, batch):
    p = params
    spad = (jnp.arange(EPT_PAD - EPT, dtype=jnp.int32) % N)[None, None, :]
    srcs = jnp.concatenate(
        [jnp.stack([edge_index[0], edge_index_t[0]]).reshape(2, NS, EPT),
         jnp.broadcast_to(spad, (2, NS, EPT_PAD - EPT))],
        axis=2).reshape(2, NS, NBLK, KCH, C)
    dpad = (N + jnp.arange(EPT_PAD - EPT, dtype=jnp.int32) % (NP - N))[None, None, :]
    dsts = jnp.concatenate(
        [jnp.stack([edge_index[1], edge_index_t[1]]).reshape(2, NS, EPT),
         jnp.broadcast_to(dpad, (2, NS, EPT_PAD - EPT))],
        axis=2).reshape(2, NS, NBLK, KCH, C)
    batch2d = batch.reshape(1, N)

    def conv_args(lp, bnp, l2p):
        return (lp["w"], _row(lp["b"]), _row(bnp["g"]), _row(bnp["b"]),
                l2p["w"], _row(l2p["b"]))

    agg = _make_agg_kernel()
    agg1 = agg(x, srcs, dsts)
    h1 = _dense12(x, agg1, *conv_args(p["c1l1"], p["c1bn"], p["c1l2"]),
                  p["fc1"]["w"][:H], p["fc1"]["w"][H:], _row(p["fc1"]["b"]))
    agg2 = agg(h1, srcs, dsts)
    h2 = _dense12(h1, agg2, *conv_args(p["c2l1"], p["c2bn"], p["c2l2"]),
                  p["fc2"]["w"][:H], p["fc2"]["w"][H:], _row(p["fc2"]["b"]))
    agg3 = agg(h2, srcs, dsts)
    zg, h = _dense3(h1, h2, agg3, batch2d,
                    *conv_args(p["c3l1"], p["c3bn"], p["c3l2"]),
                    p["fc3"]["w"][:Z], p["fc3"]["w"][Z:], _row(p["fc3"]["b"]),
                    p["fc4"]["w"][:H], p["fc4"]["w"][H:2 * H], p["fc4"]["w"][2 * H:],
                    _row(p["fc4"]["b"]),
                    p["fc5"]["w"][:H], p["fc5"]["w"][H:2 * H], p["fc5"]["w"][2 * H:],
                    _row(p["fc5"]["b"]))
    return (zg, h)


# speedup vs baseline: 1.0616x; 1.0616x over previous
import functools

import jax
import jax.numpy as jnp
from jax import lax
from jax.experimental import pallas as pl
from jax.experimental.pallas import tpu as pltpu
from jax.experimental.pallas import tpu_sc as plsc

N = 10000
E = 320000
D = 128
H = 128
Z = 64
G = 64

NC = 2
NS = 16
EPT = E // NS
C = 32
NB = 5
KCH = 20
EPT_PAD = 20480
NBLK = EPT_PAD // (C * KCH)
NP = 10240
RPT = NP // NS


def _agg_body(feat, srcs, dsts, out, idx_sa, idx_da, idx_sb, idx_db,
              r0, r1, r2, r3, r4, acc,
              g0, g1, g2, g3, g4,
              t0, t1, t2, t3, t4,
              semia, semib):
    c = lax.axis_index("c")
    s = lax.axis_index("s")
    rows = (r0, r1, r2, r3, r4)
    semg = (g0, g1, g2, g3, g4)
    sems = (t0, t1, t2, t3, t4)

    def zrow(i, carry):
        for j in range(H // 16):
            r0[i, pl.ds(16 * j, 16)] = jnp.zeros((16,), jnp.float32)
        return carry

    lax.fori_loop(0, C, zrow, 0)

    def zchunk(k, carry):
        pltpu.sync_copy(r0, acc.at[pl.ds(s * RPT + k * C, C)])
        return carry

    lax.fori_loop(0, RPT // C, zchunk, 0)
    plsc.subcore_barrier()

    def wait_gather(buf, sem):
        pltpu.make_async_copy(feat.at[idx_sa.at[0]], buf, sem).wait()

    def wait_scatter(buf, sem):
        pltpu.make_async_copy(buf, acc.at[idx_da.at[0]], sem).wait()

    def load_idx(kk, bufs, sem):
        pltpu.async_copy(srcs.at[c, s, kk], bufs[0], sem)
        pltpu.async_copy(dsts.at[c, s, kk], bufs[1], sem)

    def wait_idx(bufs, sem):
        pltpu.make_async_copy(srcs.at[c, s, 0], bufs[0], sem).wait()
        pltpu.make_async_copy(dsts.at[c, s, 0], bufs[1], sem).wait()

    def process_block(cur, nxt, nxt_sem):
        cs, cd = cur
        def quad(q, carry2):
            k = NB * q
            for j in range(NB):
                wait_gather(rows[j], semg[j])
                pltpu.async_copy(rows[j], acc.at[cd.at[k + j]], sems[j], add=True)
            for j in range(NB):
                wait_scatter(rows[j], sems[j])
                pltpu.async_copy(feat.at[cs.at[k + NB + j]], rows[j], semg[j])
            return carry2

        lax.fori_loop(0, KCH // NB - 1, quad, 0)
        k = KCH - NB
        for j in range(NB):
            wait_gather(rows[j], semg[j])
            pltpu.async_copy(rows[j], acc.at[cd.at[k + j]], sems[j], add=True)
        if nxt is None:
            for j in range(NB):
                wait_scatter(rows[j], sems[j])
        else:
            wait_idx(nxt, nxt_sem)
            for j in range(NB):
                wait_scatter(rows[j], sems[j])
                pltpu.async_copy(feat.at[nxt[0].at[j]], rows[j], semg[j])

    pltpu.sync_copy(srcs.at[c, s, 0], idx_sa)
    pltpu.sync_copy(dsts.at[c, s, 0], idx_da)
    for j in range(NB):
        pltpu.async_copy(feat.at[idx_sa.at[j]], rows[j], semg[j])
    buf_a = (idx_sa, idx_da)
    buf_b = (idx_sb, idx_db)

    def superblock(sb, carry):
        kk = 2 * sb
        load_idx(kk + 1, buf_b, semib)
        process_block(buf_a, buf_b, semib)
        load_idx(kk + 2, buf_a, semia)
        process_block(buf_b, buf_a, semia)
        return carry

    lax.fori_loop(0, NBLK // 2 - 1, superblock, 0)
    load_idx(NBLK - 1, buf_b, semib)
    process_block(buf_a, buf_b, semib)
    process_block(buf_b, None, None)
    plsc.subcore_barrier()

    pltpu.sync_copy(acc.at[pl.ds(s * RPT, RPT)], out.at[c, pl.ds(s * RPT, RPT)])


@functools.cache
def _make_agg_kernel():
    return functools.partial(
        pl.kernel,
        out_type=jax.ShapeDtypeStruct((2, NP, H), jnp.float32),
        mesh=plsc.VectorSubcoreMesh(core_axis_name="c", subcore_axis_name="s"),
        scratch_types=[
            pltpu.VMEM((KCH, C), jnp.int32),
            pltpu.VMEM((KCH, C), jnp.int32),
            pltpu.VMEM((KCH, C), jnp.int32),
            pltpu.VMEM((KCH, C), jnp.int32),
            *[pltpu.VMEM((C, H), jnp.float32) for _ in range(NB)],
            pltpu.VMEM_SHARED((NP, H), jnp.float32),
            *[pltpu.SemaphoreType.DMA for _ in range(2 * NB + 2)],
        ],
    )(_agg_body)


def _gin_mlp(f, a, w1, b1, g, bb, w2, b2, final_relu):
    h = f + a
    h = jnp.dot(h, w1, preferred_element_type=jnp.float32) + b1
    m = jnp.mean(h, axis=0, keepdims=True)
    v = jnp.mean((h - m) ** 2, axis=0, keepdims=True)
    h = (h - m) * lax.rsqrt(v + 1e-5) * g + bb
    h = jnp.maximum(h, 0.0)
    h = jnp.dot(h, w2, preferred_element_type=jnp.float32) + b2
    if final_relu:
        h = jnp.maximum(h, 0.0)
    return h


def _dense12_body(feat, agg, w1, b1, g, bb, w2, b2, fwa, fwb, fb, out):
    f = feat[...]
    ha = _gin_mlp(f, agg[0, :N], w1[...], b1[...], g[...], bb[...], w2[...], b2[...], True)
    hb = _gin_mlp(f, agg[1, :N], w1[...], b1[...], g[...], bb[...], w2[...], b2[...], True)
    h = (jnp.dot(ha, fwa[...], preferred_element_type=jnp.float32)
         + jnp.dot(hb, fwb[...], preferred_element_type=jnp.float32) + fb[...])
    out[...] = jnp.maximum(h, 0.0)


_dense12 = pl.pallas_call(
    _dense12_body,
    out_shape=jax.ShapeDtypeStruct((N, H), jnp.float32),
)


def _dense3_body(h1r, h2r, agg, batch, w1, b1, g, bb, w2, b2,
                 f3a, f3b, f3bb, f4a, f4b, f4c, f4bb, f5a, f5b, f5c, f5bb,
                 zg_out, h_out):
    h1 = h1r[...]
    h2 = h2r[...]
    ha = _gin_mlp(h2, agg[0, :N], w1[...], b1[...], g[...], bb[...], w2[...], b2[...], False)
    hb = _gin_mlp(h2, agg[1, :N], w1[...], b1[...], g[...], bb[...], w2[...], b2[...], False)
    h3 = jnp.maximum(
        jnp.dot(ha, f3a[...], preferred_element_type=jnp.float32)
        + jnp.dot(hb, f3b[...], preferred_element_type=jnp.float32) + f3bb[...], 0.0)

    sel = (lax.broadcasted_iota(jnp.int32, (G, N), 0) == batch[...]).astype(jnp.float32)
    p1 = jnp.dot(sel, h1, preferred_element_type=jnp.float32)
    p2 = jnp.dot(sel, h2, preferred_element_type=jnp.float32)
    p3 = jnp.dot(sel, h3, preferred_element_type=jnp.float32)
    zg = (jnp.dot(jnp.maximum(p1, 0.0), f4a[...], preferred_element_type=jnp.float32)
          + jnp.dot(jnp.maximum(p2, 0.0), f4b[...], preferred_element_type=jnp.float32)
          + jnp.dot(jnp.maximum(p3, 0.0), f4c[...], preferred_element_type=jnp.float32)
          + f4bb[...])
    zg_out[...] = zg
    h_out[...] = (jnp.dot(h1, f5a[...], preferred_element_type=jnp.float32)
                  + jnp.dot(h2, f5b[...], preferred_element_type=jnp.float32)
                  + jnp.dot(h3, f5c[...], preferred_element_type=jnp.float32)
                  + f5bb[...])


_dense3 = pl.pallas_call(
    _dense3_body,
    out_shape=(jax.ShapeDtypeStruct((G, Z), jnp.float32),
               jax.ShapeDtypeStruct((N, Z), jnp.float32)),
)


def _row(v):
    return v.reshape(1, -1)


def kernel(x, params, edge_index, edge_index_t, batch):
    p = params
    spad = (jnp.arange(EPT_PAD - EPT, dtype=jnp.int32) % N)[None, None, :]
    srcs = jnp.concatenate(
        [jnp.stack([edge_index[0], edge_index_t[0]]).reshape(2, NS, EPT),
         jnp.broadcast_to(spad, (2, NS, EPT_PAD - EPT))],
        axis=2).reshape(2, NS, NBLK, KCH, C)
    dpad = (N + jnp.arange(EPT_PAD - EPT, dtype=jnp.int32) % (NP - N))[None, None, :]
    dsts = jnp.concatenate(
        [jnp.stack([edge_index[1], edge_index_t[1]]).reshape(2, NS, EPT),
         jnp.broadcast_to(dpad, (2, NS, EPT_PAD - EPT))],
        axis=2).reshape(2, NS, NBLK, KCH, C)
    batch2d = batch.reshape(1, N)

    def conv_args(lp, bnp, l2p):
        return (lp["w"], _row(lp["b"]), _row(bnp["g"]), _row(bnp["b"]),
                l2p["w"], _row(l2p["b"]))

    agg = _make_agg_kernel()
    agg1 = agg(x, srcs, dsts)
    h1 = _dense12(x, agg1, *conv_args(p["c1l1"], p["c1bn"], p["c1l2"]),
                  p["fc1"]["w"][:H], p["fc1"]["w"][H:], _row(p["fc1"]["b"]))
    agg2 = agg(h1, srcs, dsts)
    h2 = _dense12(h1, agg2, *conv_args(p["c2l1"], p["c2bn"], p["c2l2"]),
                  p["fc2"]["w"][:H], p["fc2"]["w"][H:], _row(p["fc2"]["b"]))
    agg3 = agg(h2, srcs, dsts)
    zg, h = _dense3(h1, h2, agg3, batch2d,
                    *conv_args(p["c3l1"], p["c3bn"], p["c3l2"]),
                    p["fc3"]["w"][:Z], p["fc3"]["w"][Z:], _row(p["fc3"]["b"]),
                    p["fc4"]["w"][:H], p["fc4"]["w"][H:2 * H], p["fc4"]["w"][2 * H:],
                    _row(p["fc4"]["b"]),
                    p["fc5"]["w"][:H], p["fc5"]["w"][H:2 * H], p["fc5"]["w"][2 * H:],
                    _row(p["fc5"]["b"]))
    return (zg, h)
